# baseline (speedup 1.0000x reference)
import jax
import jax.numpy as jnp
from jax.experimental import pallas as pl

B, C, H, W, M, K = 2, 64, 224, 224, 8192, 9


def _conv(x, w, b):
    y = jax.lax.conv_general_dilated(x, w, window_strides=(1, 1), padding='SAME',
                                     dimension_numbers=('NCHW', 'OIHW', 'NCHW'))
    return y + b[None, :, None, None]


def _linear(x, w, b):
    return x @ w.T + b


def kernel(d_feat, r_feat, masks, w_d0, b_d0, w_d1, b_d1, w_d2, b_d2, w_r0, b_r0, w_r1, b_r1, w_r2, b_r2, w_affd, b_affd, w_affr, b_affr, w_fc1, b_fc1, w_fc3, b_fc3, w_fc4, b_fc4, w_fc5, b_fc5, w_fc6, b_fc6, locs, nnidxs):
    p = dict(w_d0=w_d0, b_d0=b_d0, w_d1=w_d1, b_d1=b_d1, w_d2=w_d2, b_d2=b_d2,
             w_r0=w_r0, b_r0=b_r0, w_r1=w_r1, b_r1=b_r1, w_r2=w_r2, b_r2=b_r2,
             w_affd=w_affd, b_affd=b_affd, w_affr=w_affr, b_affr=b_affr,
             w_fc1=w_fc1, b_fc1=b_fc1, w_fc3=w_fc3, b_fc3=b_fc3,
             w_fc4=w_fc4, b_fc4=b_fc4, w_fc5=w_fc5, b_fc5=b_fc5,
             w_fc6=w_fc6, b_fc6=b_fc6)
    d0 = jax.nn.relu(_conv(d_feat, p['w_d0'], p['b_d0']))
    d1 = _conv(d_feat, p['w_d1'], p['b_d1'])
    r0 = jax.nn.relu(_conv(r_feat, p['w_r0'], p['b_r0']))
    r1 = _conv(r_feat, p['w_r1'], p['b_r1'])
    gather = jax.vmap(lambda im, loc: im[:, loc[:, 0], loc[:, 1]].T)
    d_dis = gather(d0, locs)
    r_dis = gather(r0, locs)

    def aff(f, idx, w, b):
        kf = jnp.take(f, idx.reshape(-1), axis=0).reshape(M, K * C)
        kf = jnp.concatenate([kf, f], axis=1)
        return jax.nn.relu(_linear(kf, w, b))

    d_new = jax.vmap(lambda f, idx: aff(f, idx, p['w_affd'], p['b_affd']))(d_dis, nnidxs)
    r_new = jax.vmap(lambda f, idx: aff(f, idx, p['w_affr'], p['b_affr']))(r_dis, nnidxs)

    fuse = jax.nn.relu(_linear(jnp.concatenate([d_new, r_new], axis=-1), p['w_fc1'], p['b_fc1']))
    att1 = jax.nn.sigmoid(_linear(fuse, p['w_fc3'], p['b_fc3']))
    att2 = jax.nn.sigmoid(_linear(fuse, p['w_fc4'], p['b_fc4']))
    imfeat = d_new + r_new * att1
    ptfeat = r_new + d_new * att2
    d_new = jax.nn.relu(_linear(imfeat, p['w_fc5'], p['b_fc5']))
    r_new = jax.nn.relu(_linear(ptfeat, p['w_fc6'], p['b_fc6']))

    g = locs[:, :, 0] * W + locs[:, :, 1]
    mids = jnp.broadcast_to(jnp.arange(M, dtype=jnp.int32), (B, M))
    win = jnp.full((B, H * W), -1, jnp.int32)
    win = jax.vmap(lambda wv, gv, mv: wv.at[gv].max(mv))(win, g, mids)
    kept = (jax.vmap(lambda wv, gv: wv[gv])(win, g) == mids).astype(jnp.float32)

    def scat(f, gv, keptv):
        img = jnp.zeros((H * W, C), jnp.float32)
        img = img.at[gv].add(f * keptv[:, None])
        return img.reshape(H, W, C).transpose(2, 0, 1)

    d_sc = jax.vmap(scat)(d_new, g, kept)
    r_sc = jax.vmap(scat)(r_new, g, kept)

    d0c = (1.0 - masks) * d0 + d_sc
    r0c = (1.0 - masks) * r0 + r_sc
    d2 = _conv(d0c, p['w_d2'], p['b_d2'])
    r2 = _conv(r0c, p['w_r2'], p['b_r2'])
    return (jax.nn.relu(d2 + d1), jax.nn.relu(r2 + r1))


# speedup vs baseline: 1.0921x; 1.0921x over previous
import functools
import jax
import jax.numpy as jnp
from jax.experimental import pallas as pl

B, C, H, W, M, K = 2, 64, 224, 224, 8192, 9
HP, WP = H + 2, W + 8
HW = H * W
TH = 28
T = H // TH
ROWS = (T + 1) * TH


def _taps(xa_ref, xb_ref, w_ref, nout):
    xcat = jnp.concatenate([xa_ref[0], xb_ref[0, :3]], axis=0).reshape(-1, C)
    nrow = TH * WP
    y = jnp.zeros((nrow, nout), jnp.float32)
    for dy in range(3):
        for dx in range(3):
            off = dy * WP + dx
            y = y + jnp.dot(xcat[off:off + nrow, :],
                            w_ref[dy * 3 + dx],
                            preferred_element_type=jnp.float32)
    return y.reshape(TH, WP, nout)[:, :W, :]


def _conv_pair_body(xa_ref, xb_ref, w_ref, b_ref, m1_ref, x0_ref, x1_ref, base_ref):
    y3 = _taps(xa_ref, xb_ref, w_ref, 2 * C)
    x0 = jax.nn.relu(y3[:, :, :C] + b_ref[0, :C])
    x1 = y3[:, :, C:] + b_ref[0, C:]
    x0_ref[0] = x0
    x1_ref[0] = x1
    base_ref[0] = m1_ref[0] * x0


def _conv_pair(xp, w9, b2, m1):
    out = jax.ShapeDtypeStruct((B, H, W, C), jnp.float32)
    return pl.pallas_call(
        _conv_pair_body,
        grid=(B, T),
        in_specs=[
            pl.BlockSpec((1, TH, WP, C), lambda b, t: (b, t, 0, 0)),
            pl.BlockSpec((1, TH, WP, C), lambda b, t: (b, t + 1, 0, 0)),
            pl.BlockSpec((9, C, 2 * C), lambda b, t: (0, 0, 0)),
            pl.BlockSpec((1, 2 * C), lambda b, t: (0, 0)),
            pl.BlockSpec((1, TH, W, C), lambda b, t: (b, t, 0, 0)),
        ],
        out_specs=[
            pl.BlockSpec((1, TH, W, C), lambda b, t: (b, t, 0, 0)),
            pl.BlockSpec((1, TH, W, C), lambda b, t: (b, t, 0, 0)),
            pl.BlockSpec((1, TH, W, C), lambda b, t: (b, t, 0, 0)),
        ],
        out_shape=[out, out, out],
    )(xp, xp, w9, b2, m1)


def _conv_final_body(xa_ref, xb_ref, w_ref, b_ref, x1_ref, o_ref):
    y3 = _taps(xa_ref, xb_ref, w_ref, C)
    o_ref[0] = jax.nn.relu(y3 + b_ref[0] + x1_ref[0])


def _conv_final(xp, w9, b1, x1):
    return pl.pallas_call(
        _conv_final_body,
        grid=(B, T),
        in_specs=[
            pl.BlockSpec((1, TH, WP, C), lambda b, t: (b, t, 0, 0)),
            pl.BlockSpec((1, TH, WP, C), lambda b, t: (b, t + 1, 0, 0)),
            pl.BlockSpec((9, C, C), lambda b, t: (0, 0, 0)),
            pl.BlockSpec((1, C), lambda b, t: (0, 0)),
            pl.BlockSpec((1, TH, W, C), lambda b, t: (b, t, 0, 0)),
        ],
        out_specs=pl.BlockSpec((1, TH, W, C), lambda b, t: (b, t, 0, 0)),
        out_shape=jax.ShapeDtypeStruct((B, H, W, C), jnp.float32),
    )(xp, xp, w9, b1, x1)


def _pad_flat(x_nhwc):
    return jnp.pad(x_nhwc, ((0, 0), (1, ROWS - H - 1), (1, WP - W - 1), (0, 0)))


def _w9(w_oihw):
    return w_oihw.transpose(2, 3, 1, 0).reshape(9, C, -1)


def kernel(d_feat, r_feat, masks, w_d0, b_d0, w_d1, b_d1, w_d2, b_d2, w_r0, b_r0, w_r1, b_r1, w_r2, b_r2, w_affd, b_affd, w_affr, b_affr, w_fc1, b_fc1, w_fc3, b_fc3, w_fc4, b_fc4, w_fc5, b_fc5, w_fc6, b_fc6, locs, nnidxs):
    d_nhwc = d_feat.transpose(0, 2, 3, 1)
    r_nhwc = r_feat.transpose(0, 2, 3, 1)
    m1 = jnp.broadcast_to((1.0 - masks).transpose(0, 2, 3, 1), (B, H, W, C))

    wd = jnp.concatenate([_w9(w_d0), _w9(w_d1)], axis=-1)
    wr = jnp.concatenate([_w9(w_r0), _w9(w_r1)], axis=-1)
    bd = jnp.concatenate([b_d0, b_d1])[None, :]
    br = jnp.concatenate([b_r0, b_r1])[None, :]

    d0, d1, base_d = _conv_pair(_pad_flat(d_nhwc), wd, bd, m1)
    r0, r1, base_r = _conv_pair(_pad_flat(r_nhwc), wr, br, m1)

    g = locs[:, :, 0] * W + locs[:, :, 1]
    d0f = d0.reshape(B, HW, C)
    r0f = r0.reshape(B, HW, C)
    d_dis = jax.vmap(lambda im, gv: im[gv])(d0f, g)
    r_dis = jax.vmap(lambda im, gv: im[gv])(r0f, g)

    def aff(f, idx, w, b):
        kf = jnp.take(f, idx.reshape(-1), axis=0).reshape(M, K * C)
        kf = jnp.concatenate([kf, f], axis=1)
        return jax.nn.relu(kf @ w.T + b)

    d_new = jax.vmap(lambda f, idx: aff(f, idx, w_affd, b_affd))(d_dis, nnidxs)
    r_new = jax.vmap(lambda f, idx: aff(f, idx, w_affr, b_affr))(r_dis, nnidxs)

    fuse = jax.nn.relu(jnp.concatenate([d_new, r_new], axis=-1) @ w_fc1.T + b_fc1)
    att1 = jax.nn.sigmoid(fuse @ w_fc3.T + b_fc3)
    att2 = jax.nn.sigmoid(fuse @ w_fc4.T + b_fc4)
    imfeat = d_new + r_new * att1
    ptfeat = r_new + d_new * att2
    d_new = jax.nn.relu(imfeat @ w_fc5.T + b_fc5)
    r_new = jax.nn.relu(ptfeat @ w_fc6.T + b_fc6)

    mids = jnp.broadcast_to(jnp.arange(M, dtype=jnp.int32), (B, M))
    win = jnp.full((B, HW), -1, jnp.int32)
    win = jax.vmap(lambda wv, gv, mv: wv.at[gv].max(mv))(win, g, mids)
    kept = (jax.vmap(lambda wv, gv: wv[gv])(win, g) == mids).astype(jnp.float32)

    def scat(f, gv, keptv):
        img = jnp.zeros((HW, C), jnp.float32)
        return img.at[gv].add(f * keptv[:, None])

    delta_d = jax.vmap(scat)(d_new, g, kept)
    delta_r = jax.vmap(scat)(r_new, g, kept)

    xb_d = (base_d.reshape(B, HW, C) + delta_d).reshape(B, H, W, C)
    xb_r = (base_r.reshape(B, HW, C) + delta_r).reshape(B, H, W, C)

    out_d = _conv_final(_pad_flat(xb_d), _w9(w_d2), b_d2[None, :], d1)
    out_r = _conv_final(_pad_flat(xb_r), _w9(w_r2), b_r2[None, :], r1)
    return (out_d.transpose(0, 3, 1, 2), out_r.transpose(0, 3, 1, 2))


# speedup vs baseline: 6.0909x; 5.5771x over previous
import functools
import jax
import jax.numpy as jnp
from jax import lax
from jax.experimental import pallas as pl
from jax.experimental.pallas import tpu as pltpu
from jax.experimental.pallas import tpu_sc as plsc

B, C, H, W, M, K = 2, 64, 224, 224, 8192, 9
C2 = 2 * C
NC, NS = 2, 16
NW = NC * NS
HW = H * W
WP = W + 8
TH = 8
T = H // TH
ROWS = (T + 1) * TH


def _taps(xa_ref, xb_ref, w_ref, y):
    xcat = jnp.concatenate([xa_ref[0], xb_ref[0, :3]], axis=0).reshape(-1, C)
    nrow = TH * WP
    for dy in range(3):
        for dx in range(3):
            off = dy * WP + dx
            y = y + jnp.dot(xcat[off:off + nrow, :], w_ref[dy * 3 + dx],
                            preferred_element_type=jnp.float32)
    return y


def _conv_quad_body(da_ref, db_ref, ra_ref, rb_ref, wd_ref, wr_ref, b_ref,
                    m1_ref, dr0_ref, dr1_ref, base_ref):
    nrow = TH * WP
    y = jnp.zeros((nrow, 2 * C2), jnp.float32)
    y = _taps(da_ref, db_ref, wd_ref, y)
    y = _taps(ra_ref, rb_ref, wr_ref, y)
    y3 = y.reshape(TH, WP, 2 * C2)[:, :W, :] + b_ref[0]
    dr0 = jax.nn.relu(y3[:, :, :C2])
    dr0_ref[0] = dr0
    dr1_ref[0] = y3[:, :, C2:]
    base_ref[0] = m1_ref[0] * dr0


def _conv_quad(xpd, xpr, wd, wr, b4, m1):
    out = jax.ShapeDtypeStruct((B, H, W, C2), jnp.float32)
    blkA = lambda b, t: (b, t, 0, 0)
    blkB = lambda b, t: (b, t + 1, 0, 0)
    fix3 = lambda b, t: (0, 0, 0)
    return pl.pallas_call(
        _conv_quad_body,
        grid=(B, T),
        in_specs=[
            pl.BlockSpec((1, TH, WP, C), blkA),
            pl.BlockSpec((1, TH, WP, C), blkB),
            pl.BlockSpec((1, TH, WP, C), blkA),
            pl.BlockSpec((1, TH, WP, C), blkB),
            pl.BlockSpec((9, C, 2 * C2), fix3),
            pl.BlockSpec((9, C, 2 * C2), fix3),
            pl.BlockSpec((1, 2 * C2), lambda b, t: (0, 0)),
            pl.BlockSpec((1, TH, W, C2), blkA),
        ],
        out_specs=[
            pl.BlockSpec((1, TH, W, C2), blkA),
            pl.BlockSpec((1, TH, W, C2), blkA),
            pl.BlockSpec((1, TH, W, C2), blkA),
        ],
        out_shape=[out, out, out],
    )(xpd, xpd, xpr, xpr, wd, wr, b4, m1)


def _conv_final_body(xa_ref, xb_ref, w_ref, b_ref, x1_ref, o_ref):
    nrow = TH * WP
    xcat = jnp.concatenate([xa_ref[0], xb_ref[0, :3]], axis=0).reshape(-1, C2)
    y = jnp.zeros((nrow, C2), jnp.float32)
    for dy in range(3):
        for dx in range(3):
            off = dy * WP + dx
            y = y + jnp.dot(xcat[off:off + nrow, :], w_ref[dy * 3 + dx],
                            preferred_element_type=jnp.float32)
    y3 = y.reshape(TH, WP, C2)[:, :W, :]
    o_ref[0] = jax.nn.relu(y3 + b_ref[0] + x1_ref[0])


def _conv_final(xp, w9, b1, x1):
    return pl.pallas_call(
        _conv_final_body,
        grid=(B, T),
        in_specs=[
            pl.BlockSpec((1, TH, WP, C2), lambda b, t: (b, t, 0, 0)),
            pl.BlockSpec((1, TH, WP, C2), lambda b, t: (b, t + 1, 0, 0)),
            pl.BlockSpec((9, C2, C2), lambda b, t: (0, 0, 0)),
            pl.BlockSpec((1, C2), lambda b, t: (0, 0)),
            pl.BlockSpec((1, TH, W, C2), lambda b, t: (b, t, 0, 0)),
        ],
        out_specs=pl.BlockSpec((1, TH, W, C2), lambda b, t: (b, t, 0, 0)),
        out_shape=jax.ShapeDtypeStruct((B, H, W, C2), jnp.float32),
    )(xp, xp, w9, b1, x1)


def _sc_gather(table, idx, chunk):
    nidx = idx.shape[0]
    per_w = nidx // NW
    nch = per_w // chunk
    mesh = plsc.VectorSubcoreMesh(core_axis_name="c", subcore_axis_name="s")

    @functools.partial(
        pl.kernel, mesh=mesh,
        out_type=jax.ShapeDtypeStruct((nidx, C2), jnp.float32),
        scratch_types=[
            pltpu.VMEM((chunk,), jnp.int32),
            pltpu.VMEM((chunk, C2), jnp.float32),
            pltpu.SemaphoreType.DMA,
        ],
    )
    def k(table_hbm, idx_hbm, out_hbm, idx_v, rows_v, sem):
        wid = lax.axis_index("s") * NC + lax.axis_index("c")
        base = wid * per_w

        @pl.loop(0, nch)
        def _(j):
            off = base + j * chunk
            pltpu.sync_copy(idx_hbm.at[pl.ds(off, chunk)], idx_v)
            pltpu.async_copy(table_hbm.at[idx_v], rows_v, sem).wait()
            pltpu.sync_copy(rows_v, out_hbm.at[pl.ds(off, chunk)])

    return k(table, idx)


def _mlp_body(kf_ref, dr_ref, waff_ref, wself_ref, baff_ref,
              w1_ref, b1_ref, w34_ref, b34_ref, w56_ref, b56_ref, o_ref):
    dot = functools.partial(jnp.dot, preferred_element_type=jnp.float32)
    dr_new = jax.nn.relu(dot(kf_ref[...], waff_ref[...])
                         + dot(dr_ref[...], wself_ref[...]) + baff_ref[...])
    fuse = jax.nn.relu(dot(dr_new, w1_ref[...]) + b1_ref[...])
    att = jax.nn.sigmoid(dot(fuse, w34_ref[...]) + b34_ref[...])
    att_pack = jnp.concatenate(
        [jnp.broadcast_to(att[:, 0:1], att.shape[:1] + (C,)),
         jnp.broadcast_to(att[:, 1:2], att.shape[:1] + (C,))], axis=1)
    dr_sw = jnp.concatenate([dr_new[:, C:], dr_new[:, :C]], axis=1)
    impt = dr_new + dr_sw * att_pack
    o_ref[...] = jax.nn.relu(dot(impt, w56_ref[...]) + b56_ref[...])


def _mlp(kf, dr_dis, waff, wself, baff, w1, b1, w34, b34, w56, b56, TM=2048):
    n = dr_dis.shape[0]
    row = lambda i: (i, 0)
    fix = lambda i: (0, 0)
    return pl.pallas_call(
        _mlp_body,
        grid=(n // TM,),
        in_specs=[
            pl.BlockSpec((TM, K * C2), row), pl.BlockSpec((TM, C2), row),
            pl.BlockSpec((K * C2, C2), fix), pl.BlockSpec((C2, C2), fix),
            pl.BlockSpec((1, C2), fix),
            pl.BlockSpec((C2, C), fix), pl.BlockSpec((1, C), fix),
            pl.BlockSpec((C, 2), fix), pl.BlockSpec((1, 2), fix),
            pl.BlockSpec((C2, C2), fix), pl.BlockSpec((1, C2), fix),
        ],
        out_specs=pl.BlockSpec((TM, C2), row),
        out_shape=jax.ShapeDtypeStruct((n, C2), jnp.float32),
    )(kf, dr_dis, waff, wself, baff, w1, b1, w34, b34, w56, b56)


def _pad_flat(x_nhwc):
    return jnp.pad(x_nhwc, ((0, 0), (1, ROWS - H - 1), (1, WP - W - 1), (0, 0)))


def _w9(w_oihw):
    return w_oihw.transpose(2, 3, 1, 0).reshape(9, C, -1)


def _blkdiag(a, b):
    ka, na = a.shape
    kb, nb = b.shape
    z = jnp.zeros((ka + kb, na + nb), a.dtype)
    return z.at[:ka, :na].set(a).at[ka:, na:].set(b)


def kernel(d_feat, r_feat, masks, w_d0, b_d0, w_d1, b_d1, w_d2, b_d2, w_r0, b_r0, w_r1, b_r1, w_r2, b_r2, w_affd, b_affd, w_affr, b_affr, w_fc1, b_fc1, w_fc3, b_fc3, w_fc4, b_fc4, w_fc5, b_fc5, w_fc6, b_fc6, locs, nnidxs):
    d_nhwc = d_feat.transpose(0, 2, 3, 1)
    r_nhwc = r_feat.transpose(0, 2, 3, 1)
    m1 = jnp.broadcast_to((1.0 - masks).transpose(0, 2, 3, 1), (B, H, W, C2))

    zc = jnp.zeros((9, C, C), jnp.float32)
    wd = jnp.concatenate([_w9(w_d0), zc, _w9(w_d1), zc], axis=-1)
    wr = jnp.concatenate([zc, _w9(w_r0), zc, _w9(w_r1)], axis=-1)
    b4 = jnp.concatenate([b_d0, b_r0, b_d1, b_r1])[None, :]

    dr0, dr1, base = _conv_quad(_pad_flat(d_nhwc), _pad_flat(r_nhwc),
                                wd, wr, b4, m1)

    g = locs[:, :, 0].astype(jnp.int32) * W + locs[:, :, 1].astype(jnp.int32)
    gid = (g + jnp.arange(B, dtype=jnp.int32)[:, None] * HW).reshape(-1)
    nng = (nnidxs.astype(jnp.int32)
           + jnp.arange(B, dtype=jnp.int32)[:, None, None] * M).reshape(-1)

    dr_dis = _sc_gather(dr0.reshape(B * HW, C2), gid, 512)
    kf = _sc_gather(dr_dis, nng, 512).reshape(B * M, K * C2)

    wa3 = _w9_aff(w_affd, w_affr)
    wself = _blkdiag(w_affd.T[K * C:], w_affr.T[K * C:])
    baff = jnp.concatenate([b_affd, b_affr])[None, :]
    w34 = jnp.concatenate([w_fc3, w_fc4], axis=0).T
    b34 = jnp.concatenate([b_fc3, b_fc4])[None, :]
    w56 = _blkdiag(w_fc5.T, w_fc6.T)
    b56 = jnp.concatenate([b_fc5, b_fc6])[None, :]

    fdr = _mlp(kf, dr_dis, wa3, wself, baff, w_fc1.T, b_fc1[None, :],
               w34, b34, w56, b56)

    f3 = fdr.reshape(B, M, C2)
    mids = jnp.broadcast_to(jnp.arange(M, dtype=jnp.int32), (B, M))
    win = jnp.full((B, HW), -1, jnp.int32)
    win = jax.vmap(lambda wv, gv, mv: wv.at[gv].max(mv))(win, g, mids)
    kept = (jax.vmap(lambda wv, gv: wv[gv])(win, g) == mids).astype(jnp.float32)

    def scat(f, gv, keptv):
        img = jnp.zeros((HW, C2), jnp.float32)
        return img.at[gv].add(f * keptv[:, None])

    delta = jax.vmap(scat)(f3, g, kept)

    xb = (base.reshape(B, HW, C2) + delta).reshape(B, H, W, C2)

    w2 = jnp.zeros((9, C2, C2), jnp.float32)
    w2 = w2.at[:, :C, :C].set(_w9(w_d2)).at[:, C:, C:].set(_w9(w_r2))
    b2 = jnp.concatenate([b_d2, b_r2])[None, :]
    out = _conv_final(_pad_flat(xb), w2, b2, dr1)
    return (out[..., :C].transpose(0, 3, 1, 2),
            out[..., C:].transpose(0, 3, 1, 2))


def _w9_aff(w_affd, w_affr):
    wad = w_affd.T[:K * C].reshape(K, C, C)
    war = w_affr.T[:K * C].reshape(K, C, C)
    z = jnp.zeros((K, C, C), jnp.float32)
    top = jnp.concatenate([wad, z], axis=-1)
    bot = jnp.concatenate([z, war], axis=-1)
    return jnp.concatenate([top, bot], axis=1).reshape(K * C2, C2)
